# baseline (speedup 1.0000x reference)
import jax
import jax.numpy as jnp
from jax.experimental import pallas as pl
from jax.experimental.pallas import tpu as pltpu

_BJ = 512


def _fused_kernel(g_ref, x_ref, ws1_ref, wn1_ref, b1_ref, ws2_ref, wn2_ref,
                  b2_ref, out_ref, gbf_ref, xaugt_ref, h1_ref, h1t_ref,
                  inv_ref):
    l = pl.program_id(0)
    j = pl.program_id(1)
    n, d_in = x_ref.shape

    @pl.when(l == 0)
    def _layer0():
        @pl.when(j == 0)
        def _stage_x():
            xaugt_ref[:d_in, :] = jnp.transpose(
                x_ref[...]).astype(jnp.bfloat16)
            xaugt_ref[d_in:, :] = jnp.ones((1, n), jnp.bfloat16)

        gb = g_ref[...].astype(jnp.bfloat16)
        gbf_ref[:, pl.ds(j * _BJ, _BJ)] = gb
        acct = jax.lax.dot_general(
            xaugt_ref[...], gb, (((1,), (0,)), ((), ())),
            preferred_element_type=jnp.float32)
        inv = 1.0 / jnp.maximum(acct[d_in:, :], 1.0)
        inv_ref[:, pl.ds(j * _BJ, _BJ)] = inv
        neight = (acct[:d_in, :] * inv).astype(jnp.bfloat16)
        hd = x_ref[pl.ds(j * _BJ, _BJ), :].astype(jnp.bfloat16)
        h1 = (jax.lax.dot_general(
                  hd, ws1_ref[...], (((1,), (0,)), ((), ())),
                  preferred_element_type=jnp.float32)
              + jax.lax.dot_general(
                  neight, wn1_ref[...], (((0,), (0,)), ((), ())),
                  preferred_element_type=jnp.float32)
              + b1_ref[...])
        h1b = jnp.maximum(h1, 0.0).astype(jnp.bfloat16)
        h1_ref[pl.ds(j * _BJ, _BJ), :] = h1b
        h1t_ref[:, pl.ds(j * _BJ, _BJ)] = jnp.transpose(h1b)

    @pl.when(l == 1)
    def _layer1():
        gb = gbf_ref[:, pl.ds(j * _BJ, _BJ)]
        acct = jax.lax.dot_general(
            h1t_ref[...], gb, (((1,), (0,)), ((), ())),
            preferred_element_type=jnp.float32)
        inv = inv_ref[:, pl.ds(j * _BJ, _BJ)]
        neight = (acct * inv).astype(jnp.bfloat16)
        hd = h1_ref[pl.ds(j * _BJ, _BJ), :]
        out = (jax.lax.dot_general(
                   hd, ws2_ref[...], (((1,), (0,)), ((), ())),
                   preferred_element_type=jnp.float32)
               + jax.lax.dot_general(
                   neight, wn2_ref[...], (((0,), (0,)), ((), ())),
                   preferred_element_type=jnp.float32)
               + b2_ref[...])
        out_ref[...] = out


def kernel(inputs, graph, W_self1, W_neigh1, b1, W_self2, W_neigh2, b2):
    n, d_in = inputs.shape
    d_hid = W_self1.shape[1]
    d_out = W_self2.shape[1]
    nj = n // _BJ
    ws1b = W_self1.astype(jnp.bfloat16)
    wn1b = W_neigh1.astype(jnp.bfloat16)
    ws2b = W_self2.astype(jnp.bfloat16)
    wn2b = W_neigh2.astype(jnp.bfloat16)
    return pl.pallas_call(
        _fused_kernel,
        grid=(2, nj),
        in_specs=[
            pl.BlockSpec((n, _BJ), lambda l, j: (0, jnp.where(l == 0, j, 0))),
            pl.BlockSpec((n, d_in), lambda l, j: (0, 0)),
            pl.BlockSpec((d_in, d_hid), lambda l, j: (0, 0)),
            pl.BlockSpec((d_in, d_hid), lambda l, j: (0, 0)),
            pl.BlockSpec((1, d_hid), lambda l, j: (0, 0)),
            pl.BlockSpec((d_hid, d_out), lambda l, j: (0, 0)),
            pl.BlockSpec((d_hid, d_out), lambda l, j: (0, 0)),
            pl.BlockSpec((1, d_out), lambda l, j: (0, 0)),
        ],
        out_specs=pl.BlockSpec((_BJ, d_out),
                               lambda l, j: (jnp.where(l == 0, 0, j), 0)),
        out_shape=jax.ShapeDtypeStruct((n, d_out), jnp.float32),
        scratch_shapes=[
            pltpu.VMEM((n, n), jnp.bfloat16),
            pltpu.VMEM((d_in + 1, n), jnp.bfloat16),
            pltpu.VMEM((n, d_hid), jnp.bfloat16),
            pltpu.VMEM((d_hid, n), jnp.bfloat16),
            pltpu.VMEM((1, n), jnp.float32),
        ],
        compiler_params=pltpu.CompilerParams(
            dimension_semantics=("arbitrary", "arbitrary")),
    )(graph, inputs, ws1b, wn1b, b1.reshape(1, -1), ws2b, wn2b,
      b2.reshape(1, -1))
